# baseline (speedup 1.0000x reference)
import functools

import jax
import jax.numpy as jnp
from jax import lax
from jax.experimental import pallas as pl
from jax.experimental import layout as jex_layout
from jax.experimental.pallas import tpu as pltpu
from jax.experimental.pallas import tpu_sc as plsc

POOL = 64
LEN = 16
DIM = 2048
K = 2
BATCH = 1024

SUB = 16
POSITIONS = BATCH * K
NC, NS = 2, 16
NW = NC * NS
BPW = POSITIONS // NW
BT = 256


def _route_body(q_ref, k_ref, attn_ref, idx_ref):
    q = q_ref[...]
    k = k_ref[...]
    sims = lax.dot_general(q, k, (((1,), (1,)), ((), ())),
                           preferred_element_type=jnp.float32)
    m1 = jnp.max(sims, axis=-1, keepdims=True)
    e = jnp.exp(sims - m1)
    attn_ref[...] = e / jnp.sum(e, axis=-1, keepdims=True)
    col = lax.broadcasted_iota(jnp.int32, sims.shape, 1)
    i1 = jnp.min(jnp.where(sims == m1, col, POOL), axis=-1, keepdims=True)
    sims2 = jnp.where(col == i1, -jnp.inf, sims)
    m2 = jnp.max(sims2, axis=-1, keepdims=True)
    i2 = jnp.min(jnp.where(sims2 == m2, col, POOL), axis=-1, keepdims=True)
    idx_ref[...] = jnp.concatenate([i1, i2], axis=1)


_route = pl.pallas_call(
    _route_body,
    grid=(BATCH // BT,),
    in_specs=[
        pl.BlockSpec((BT, DIM), lambda i: (i, 0)),
        pl.BlockSpec((POOL, DIM), lambda i: (0, 0)),
    ],
    out_specs=[
        pl.BlockSpec((BT, POOL), lambda i: (i, 0)),
        pl.BlockSpec((BT, K), lambda i: (i, 0)),
    ],
    out_shape=[
        jax.ShapeDtypeStruct((BATCH, POOL), jnp.float32),
        jax.ShapeDtypeStruct((BATCH, K), jnp.int32),
    ],
)


HALF = SUB // 2
HSTEPS = BPW


def _sc_gather_body(table, fidx, out, idx_v,
                    tb0, tb1, tb2, spbuf,
                    ga0, ga1, ga2, pa0, pa1, pa2,
                    gb0, gb1, gb2, pb0, pb1, pb2):
    sid = lax.axis_index("s")
    wid = sid * NC + lax.axis_index("c")
    base = wid * BPW
    pltpu.sync_copy(fidx.at[pl.ds(base, BPW)], idx_v)
    paths = (
        ((tb0, tb1, tb2), (ga0, ga1, ga2), (pa0, pa1, pa2), 0),
        (tuple(spbuf.at[sid, pl.ds(b * HALF, HALF)] for b in range(3)),
         (gb0, gb1, gb2), (pb0, pb1, pb2), HSTEPS),
    )

    def src(h):
        s = idx_v[pl.ds(h // 2, 1)][0]
        return table.at[pl.ds(s * SUB + h % 2 * HALF, HALF)]

    def dput(h):
        pos = base + h // 2
        return out.at[pos // K, pl.ds(pos % K * LEN + h % 2 * HALF, HALF)]

    for bufs, gs, ps, off in paths:
        for b in range(2):
            pltpu.async_copy(src(off + b), bufs[b], gs[b])

    def step(t, b, bufs, gs, ps, off):
        h = off + t
        pltpu.make_async_copy(src(h), bufs[b], gs[b]).wait()
        pltpu.async_copy(bufs[b], dput(h), ps[b])
        b2 = (b + 2) % 3

        @pl.when(t + 2 < HSTEPS)
        def _():
            @pl.when(t >= 1)
            def _():
                pltpu.make_async_copy(bufs[b2], dput(h - 1), ps[b2]).wait()

            pltpu.async_copy(src(h + 2), bufs[b2], gs[b2])

    def body(j, carry):
        for b in range(3):
            for path in paths:
                step(3 * j + b, b, *path)
        return carry

    lax.fori_loop(0, (HSTEPS - 1) // 3, body, 0)
    t = HSTEPS - 1
    b = t % 3
    for bufs, gs, ps, off in paths:
        pltpu.make_async_copy(src(off + t), bufs[b], gs[b]).wait()
        pltpu.async_copy(bufs[b], dput(off + t), ps[b])
    for bufs, gs, ps, off in paths:
        for back in range(3):
            t2 = HSTEPS - 1 - back
            pltpu.make_async_copy(
                bufs[t2 % 3], dput(off + t2), ps[t2 % 3]).wait()


@functools.cache
def _make_sc_gather():
    return pl.kernel(
        _sc_gather_body,
        out_type=jax.ShapeDtypeStruct((BATCH, K * LEN, DIM), jnp.float32),
        mesh=plsc.VectorSubcoreMesh(core_axis_name="c", subcore_axis_name="s",
                                    num_cores=NC, num_subcores=NS),
        scratch_types=[
            pltpu.VMEM((BPW,), jnp.int32),
            pltpu.VMEM((HALF, DIM), jnp.float32),
            pltpu.VMEM((HALF, DIM), jnp.float32),
            pltpu.VMEM((HALF, DIM), jnp.float32),
            pltpu.VMEM_SHARED((NS, 3 * HALF, DIM), jnp.float32),
        ] + [pltpu.SemaphoreType.DMA] * 12,
    )


def _kernel_impl(query, prompts, keys):
    attn, idx32 = _route(query, keys)
    table = prompts.reshape(POOL * SUB, DIM)
    fidx = idx32.reshape(POSITIONS)
    selected = _make_sc_gather()(table, fidx)
    return selected, attn


@functools.cache
def _jitted_kernel(dev):
    sharding = jax.sharding.SingleDeviceSharding(dev)
    sel_fmt = jex_layout.Format(
        jex_layout.Layout(major_to_minor=(0, 1, 2), tiling=()), sharding)
    return jax.jit(_kernel_impl, out_shardings=(sel_fmt, sharding))


def kernel(query, prompts, keys):
    return _jitted_kernel(jax.devices()[0])(query, prompts, keys)


# speedup vs baseline: 1.2392x; 1.2392x over previous
import functools

import jax
import jax.numpy as jnp
from jax import lax
from jax.experimental import pallas as pl
from jax.experimental import layout as jex_layout
from jax.experimental.pallas import tpu as pltpu
from jax.experimental.pallas import tpu_sc as plsc

POOL = 64
LEN = 16
DIM = 2048
K = 2
BATCH = 1024

SUB = 16
POSITIONS = BATCH * K
NC, NS = 2, 16
NW = NC * NS
BPW = POSITIONS // NW
BT = 256


def _route_body(q_ref, k_ref, attn_ref, idx_ref):
    q = q_ref[...]
    k = k_ref[...]
    sims = lax.dot_general(q, k, (((1,), (1,)), ((), ())),
                           preferred_element_type=jnp.float32)
    m1 = jnp.max(sims, axis=-1, keepdims=True)
    e = jnp.exp(sims - m1)
    attn_ref[...] = e / jnp.sum(e, axis=-1, keepdims=True)
    col = lax.broadcasted_iota(jnp.int32, sims.shape, 1)
    i1 = jnp.min(jnp.where(sims == m1, col, POOL), axis=-1, keepdims=True)
    sims2 = jnp.where(col == i1, -jnp.inf, sims)
    m2 = jnp.max(sims2, axis=-1, keepdims=True)
    i2 = jnp.min(jnp.where(sims2 == m2, col, POOL), axis=-1, keepdims=True)
    idx_ref[...] = jnp.concatenate([i1, i2], axis=1)


_route = pl.pallas_call(
    _route_body,
    grid=(BATCH // BT,),
    in_specs=[
        pl.BlockSpec((BT, DIM), lambda i: (i, 0)),
        pl.BlockSpec((POOL, DIM), lambda i: (0, 0)),
    ],
    out_specs=[
        pl.BlockSpec((BT, POOL), lambda i: (i, 0)),
        pl.BlockSpec((BT, K), lambda i: (i, 0)),
    ],
    out_shape=[
        jax.ShapeDtypeStruct((BATCH, POOL), jnp.float32),
        jax.ShapeDtypeStruct((BATCH, K), jnp.int32),
    ],
)


HALFPOOL = POOL // NC
HROWS = HALFPOOL * SUB
PPT = POSITIONS // NS
NSEM = 4


def _sc_gather_body(table, fidx, out, idx_v, sp_table, s0, s1, s2, s3):
    sid = lax.axis_index("s")
    cid = lax.axis_index("c")
    base = sid * PPT
    pltpu.sync_copy(fidx.at[pl.ds(base, PPT)], idx_v)
    pltpu.sync_copy(table.at[pl.ds(cid * HROWS + sid * (HROWS // NS),
                                   HROWS // NS)],
                    sp_table.at[pl.ds(sid * (HROWS // NS), HROWS // NS)])
    plsc.subcore_barrier()

    sems = (s0, s1, s2, s3)

    def refs(i):
        s = idx_v[pl.ds(i, 1)][0]
        pos = base + i
        dst = out.at[pos // K, pl.ds(pos % K * LEN, LEN)]
        return s, dst

    def issue(i, b):
        s, dst = refs(i)

        @pl.when(s // HALFPOOL == cid)
        def _():
            pltpu.async_copy(
                sp_table.at[pl.ds(s * SUB - cid * HROWS, SUB)], dst, sems[b])

    def drain(i, b):
        s, dst = refs(i)

        @pl.when(s // HALFPOOL == cid)
        def _():
            pltpu.make_async_copy(
                sp_table.at[pl.ds(s * SUB - cid * HROWS, SUB)], dst,
                sems[b]).wait()

    for b in range(NSEM):
        issue(b, b)

    def body(j, carry):
        for b in range(NSEM):
            i = NSEM * j + b
            drain(i - NSEM, b)
            issue(i, b)
        return carry

    lax.fori_loop(1, PPT // NSEM, body, 0)
    for b in range(NSEM):
        drain(PPT - NSEM + b, b)


@functools.cache
def _make_sc_gather():
    return pl.kernel(
        _sc_gather_body,
        out_type=jax.ShapeDtypeStruct((BATCH, K * LEN, DIM), jnp.float32),
        mesh=plsc.VectorSubcoreMesh(core_axis_name="c", subcore_axis_name="s",
                                    num_cores=NC, num_subcores=NS),
        scratch_types=[
            pltpu.VMEM((PPT,), jnp.int32),
            pltpu.VMEM_SHARED((HROWS, DIM), jnp.float32),
        ] + [pltpu.SemaphoreType.DMA] * NSEM,
    )


def _kernel_impl(query, prompts, keys):
    attn, idx32 = _route(query, keys)
    table = prompts.reshape(POOL * SUB, DIM)
    fidx = idx32.reshape(POSITIONS)
    selected = _make_sc_gather()(table, fidx)
    return selected, attn


@functools.cache
def _jitted_kernel(dev):
    sharding = jax.sharding.SingleDeviceSharding(dev)
    sel_fmt = jex_layout.Format(
        jex_layout.Layout(major_to_minor=(0, 1, 2), tiling=()), sharding)
    return jax.jit(_kernel_impl, out_shardings=(sel_fmt, sharding))


def kernel(query, prompts, keys):
    return _jitted_kernel(jax.devices()[0])(query, prompts, keys)


# speedup vs baseline: 1.2407x; 1.0012x over previous
import functools

import jax
import jax.numpy as jnp
from jax import lax
from jax.experimental import pallas as pl
from jax.experimental import layout as jex_layout
from jax.experimental.pallas import tpu as pltpu
from jax.experimental.pallas import tpu_sc as plsc

POOL = 64
LEN = 16
DIM = 2048
K = 2
BATCH = 1024

SUB = 16
POSITIONS = BATCH * K
NC, NS = 2, 16
BT = 256


def _route_body(q_ref, k_ref, attn_ref, idx_ref):
    q = q_ref[...]
    k = k_ref[...]
    sims = lax.dot_general(q, k, (((1,), (1,)), ((), ())),
                           preferred_element_type=jnp.float32)
    m1 = jnp.max(sims, axis=-1, keepdims=True)
    e = jnp.exp(sims - m1)
    attn_ref[...] = e / jnp.sum(e, axis=-1, keepdims=True)
    col = lax.broadcasted_iota(jnp.int32, sims.shape, 1)
    i1 = jnp.min(jnp.where(sims == m1, col, POOL), axis=-1, keepdims=True)
    sims2 = jnp.where(col == i1, -jnp.inf, sims)
    m2 = jnp.max(sims2, axis=-1, keepdims=True)
    i2 = jnp.min(jnp.where(sims2 == m2, col, POOL), axis=-1, keepdims=True)
    idx_ref[...] = jnp.concatenate([i1, i2], axis=1)


_route = pl.pallas_call(
    _route_body,
    grid=(BATCH // BT,),
    in_specs=[
        pl.BlockSpec((BT, DIM), lambda i: (i, 0)),
        pl.BlockSpec((POOL, DIM), lambda i: (0, 0)),
    ],
    out_specs=[
        pl.BlockSpec((BT, POOL), lambda i: (i, 0)),
        pl.BlockSpec((BT, K), lambda i: (i, 0)),
    ],
    out_shape=[
        jax.ShapeDtypeStruct((BATCH, POOL), jnp.float32),
        jax.ShapeDtypeStruct((BATCH, K), jnp.int32),
    ],
)


HALFPOOL = POOL // NC
HROWS = HALFPOOL * SUB
PPT = POSITIONS // NS
NSEM = 4


def _sc_gather_body(table, fidx, out, idx_v, sp_table, s0, s1, s2, s3):
    sid = lax.axis_index("s")
    cid = lax.axis_index("c")
    base = sid * PPT
    pltpu.sync_copy(fidx.at[pl.ds(base, PPT)], idx_v)
    pltpu.sync_copy(table.at[pl.ds(cid * HROWS + sid * (HROWS // NS),
                                   HROWS // NS)],
                    sp_table.at[pl.ds(sid * (HROWS // NS), HROWS // NS)])
    plsc.subcore_barrier()

    sems = (s0, s1, s2, s3)

    def refs(i):
        s = idx_v[pl.ds(i, 1)][0]
        pos = base + i
        dst = out.at[pos // K, pl.ds(pos % K * LEN, LEN)]
        return s, dst

    def issue(i, b):
        s, dst = refs(i)

        @pl.when(s // HALFPOOL == cid)
        def _():
            pltpu.async_copy(
                sp_table.at[pl.ds(s * SUB - cid * HROWS, SUB)], dst, sems[b])

    def drain(i, b):
        s, dst = refs(i)

        @pl.when(s // HALFPOOL == cid)
        def _():
            pltpu.make_async_copy(
                sp_table.at[pl.ds(s * SUB - cid * HROWS, SUB)], dst,
                sems[b]).wait()

    for b in range(NSEM):
        issue(b, b)

    def body(j, carry):
        for b in range(NSEM):
            i = NSEM * j + b
            drain(i - NSEM, b)
            issue(i, b)
        return carry

    lax.fori_loop(1, PPT // NSEM, body, 0)
    for b in range(NSEM):
        drain(PPT - NSEM + b, b)


@functools.cache
def _make_sc_gather():
    return pl.kernel(
        _sc_gather_body,
        out_type=jax.ShapeDtypeStruct((BATCH, K * LEN, DIM), jnp.float32),
        mesh=plsc.VectorSubcoreMesh(core_axis_name="c", subcore_axis_name="s",
                                    num_cores=NC, num_subcores=NS),
        scratch_types=[
            pltpu.VMEM((PPT,), jnp.int32),
            pltpu.VMEM_SHARED((HROWS, DIM), jnp.float32),
        ] + [pltpu.SemaphoreType.DMA] * NSEM,
    )


def _kernel_impl(query, prompts, keys):
    attn, idx32 = _route(query, keys)
    table = prompts.reshape(POOL * SUB, DIM)
    fidx = idx32.reshape(POSITIONS)
    selected = _make_sc_gather()(table, fidx)
    return selected, attn


@functools.cache
def _jitted_kernel(dev):
    sharding = jax.sharding.SingleDeviceSharding(dev)
    sel_fmt = jex_layout.Format(
        jex_layout.Layout(major_to_minor=(0, 1, 2), tiling=()), sharding)
    return jax.jit(_kernel_impl, out_shardings=(sel_fmt, sharding))


def kernel(query, prompts, keys):
    return _jitted_kernel(jax.devices()[0])(query, prompts, keys)
